# baseline (speedup 1.0000x reference)
import functools

import jax
import jax.numpy as jnp
from jax import lax
from jax.experimental import pallas as pl
from jax.experimental.pallas import tpu as pltpu
from jax.experimental.pallas import tpu_sc as plsc

B = 16384
N_COLS = 26
VOCAB = 100000
DIM = 16

NUM_CORES = 2
NUM_SUBCORES = 16
CPC = N_COLS // NUM_CORES
BT = B // 128
TPW = CPC * BT // NUM_SUBCORES
GPC = 8
NCHUNK = TPW // GPC
HALF = GPC // 2
VT = VOCAB // 128
NBLK = CPC * VT
P1IT = (NBLK + NUM_SUBCORES - 1) // NUM_SUBCORES
SPC = 12504


def _body(idx_hbm, tab_hbm, tail_hbm, out_hbm, slab_hbm, stage_v, srow_v,
          idx_v, sidx_v, q16_v, slabs_v, tiles_v, sem):
  core = lax.axis_index("c")
  sub = lax.axis_index("s")
  iota = lax.iota(jnp.int32, 16)

  def p1_body(i, carry):
    bid = i * NUM_SUBCORES + sub
    col_i = bid // VT
    vt = bid - col_i * VT
    c = core + 2 * col_i

    @pl.when(bid < NBLK)
    def _full():
      v0 = pl.multiple_of(vt * 128, 128)
      pltpu.sync_copy(tab_hbm.at[c].at[pl.ds(0, 8), pl.ds(v0, 128)],
                      stage_v.at[pl.ds(0, 8)])
      pltpu.sync_copy(tab_hbm.at[c].at[pl.ds(8, 8), pl.ds(v0, 128)],
                      stage_v.at[pl.ds(8, 8)])
      for sr in range(16):
        for vi in range(8):
          vals = plsc.load_gather(
              stage_v, [iota, jnp.full((16,), sr * 8 + vi, jnp.int32)])
          srow_v[sr, pl.ds(vi * 16, 16)] = vals
      s0 = pl.multiple_of(vt * 16, 16)
      pltpu.sync_copy(srow_v, slab_hbm.at[c].at[pl.ds(s0, 16)])

    return carry

  lax.fori_loop(0, P1IT, p1_body, 0)

  @pl.when(sub < CPC)
  def _tails():
    c = core + 2 * sub
    pltpu.sync_copy(tail_hbm.at[c], srow_v.at[pl.ds(0, 8)])
    pltpu.sync_copy(srow_v.at[pl.ds(0, 8)],
                    slab_hbm.at[c].at[pl.ds(VT * 16, 8)])
  plsc.subcore_barrier()

  def chunk_body(k, carry):
    t = sub * TPW + k * GPC
    col_i = t // (BT)
    c = core + 2 * col_i
    bt0 = pl.multiple_of(t - col_i * BT, GPC)
    pltpu.sync_copy(idx_hbm.at[c].at[pl.ds(bt0, GPC)], idx_v)
    for j in range(GPC):
      for l in range(8):
        v = idx_v[j, pl.ds(l * 16, 16)]
        sidx_v[j, pl.ds(l * 16, 16)] = lax.shift_right_logical(v, 3)
        q16_v[j, pl.ds(l * 16, 16)] = lax.shift_left(
            lax.bitwise_and(v, jnp.int32(7)), 4)
    for h in range(2):
      copies = []
      for jj in range(HALF):
        copies.append(
            pltpu.async_copy(
                slab_hbm.at[c].at[sidx_v.at[h * HALF + jj]],
                slabs_v.at[pl.ds(jj * 128, 128)], sem))
      for cp in copies:
        cp.wait()
      for jj in range(HALF):
        j = h * HALF + jj
        for l in range(8):
          q16s = q16_v[j, pl.ds(l * 16, 16)]
          ridx = iota + (jj * 128 + l * 16)
          for d in range(DIM):
            dt, di = d // 8, d % 8
            vals = plsc.load_gather(slabs_v, [ridx, q16s + d])
            tiles_v[dt, j, di, pl.ds(l * 16, 16)] = vals
    pltpu.sync_copy(tiles_v.at[0], out_hbm.at[c, 0].at[pl.ds(bt0, GPC)])
    pltpu.sync_copy(tiles_v.at[1], out_hbm.at[c, 1].at[pl.ds(bt0, GPC)])
    return carry

  lax.fori_loop(0, NCHUNK, chunk_body, 0)


@jax.jit
def _embed(idx3d, tab_t, tail8):
  mesh = plsc.VectorSubcoreMesh(core_axis_name="c", subcore_axis_name="s")
  f = pl.kernel(
      _body,
      mesh=mesh,
      out_type=(
          jax.ShapeDtypeStruct((N_COLS, 2, BT, 8, 128), jnp.float32),
          jax.ShapeDtypeStruct((N_COLS, SPC, 128), jnp.float32),
      ),
      scratch_types=[
          pltpu.VMEM((16, 128), jnp.float32),
          pltpu.VMEM((16, 128), jnp.float32),
          pltpu.VMEM((GPC, 128), jnp.int32),
          pltpu.VMEM((GPC, 128), jnp.int32),
          pltpu.VMEM((GPC, 128), jnp.int32),
          pltpu.VMEM((HALF * 128, 128), jnp.float32),
          pltpu.VMEM((2, GPC, 8, 128), jnp.float32),
          pltpu.SemaphoreType.DMA,
      ],
      compiler_params=pltpu.CompilerParams(
          use_tc_tiling_on_sc=True, needs_layout_passes=False),
  )
  return f(idx3d, tab_t, tail8)


def kernel(value, tables):
  tab_t = tables.transpose(0, 2, 1)
  tail = tables[:, VT * 128:, :].reshape(N_COLS, 4, 128)
  tail8 = jnp.concatenate([tail, jnp.zeros((N_COLS, 4, 128), jnp.float32)],
                          axis=1)
  idx3d = value.astype(jnp.int32).T.reshape(N_COLS, BT, 128)
  out5d, _ = _embed(idx3d, tab_t, tail8)
  return out5d.transpose(2, 4, 0, 1, 3).reshape(B, N_COLS, DIM)


# speedup vs baseline: 1.7447x; 1.7447x over previous
import functools

import jax
import jax.numpy as jnp
from jax import lax
from jax.experimental import pallas as pl
from jax.experimental.pallas import tpu as pltpu
from jax.experimental.pallas import tpu_sc as plsc

B = 16384
N_COLS = 26
VOCAB = 100000
DIM = 16

NUM_CORES = 2
NUM_SUBCORES = 16
NW = NUM_CORES * NUM_SUBCORES
BT = B // 128
NTASK = N_COLS * BT
TPW = NTASK // NW
GPC = 8
CHUNK = GPC * 128
NCHUNK = TPW // GPC
HALF = GPC // 2
NSLAB = (N_COLS * VOCAB) // 8


def _gather_body(idx_hbm, tab_hbm, out_hbm, idx_v, rows_v, tiles_v, sem):
  wid = lax.axis_index("s") * NUM_CORES + lax.axis_index("c")
  t0 = wid * TPW

  iota = lax.iota(jnp.int32, 16)
  dvec = [jnp.full((16,), d, jnp.int32) for d in range(DIM)]

  def chunk_body(k, carry):
    t = t0 + k * GPC
    c = t // BT
    bt0 = pl.multiple_of(t - c * BT, GPC)
    pltpu.sync_copy(idx_hbm.at[c].at[pl.ds(bt0, GPC)], idx_v)
    copies = []
    for j in range(GPC):
      copies.append(
          pltpu.async_copy(
              tab_hbm.at[idx_v.at[j]], rows_v.at[pl.ds(j * 128, 128)], sem))
    for cp in copies:
      cp.wait()
    for j in range(GPC):
      for d in range(DIM):
        dt, di = d // 8, d % 8
        for l in range(8):
          ridx = iota + (j * 128 + l * 16)
          vals = plsc.load_gather(rows_v, [ridx, dvec[d]])
          tiles_v[dt, j, di, pl.ds(l * 16, 16)] = vals
    pltpu.sync_copy(tiles_v.at[0], out_hbm.at[c, 0].at[pl.ds(bt0, GPC)])
    pltpu.sync_copy(tiles_v.at[1], out_hbm.at[c, 1].at[pl.ds(bt0, GPC)])
    return carry

  lax.fori_loop(0, NCHUNK, chunk_body, 0)


@jax.jit
def _embed(idx3d, flat_tab):
  mesh = plsc.VectorSubcoreMesh(core_axis_name="c", subcore_axis_name="s")
  f = pl.kernel(
      _gather_body,
      mesh=mesh,
      out_type=jax.ShapeDtypeStruct((N_COLS, 2, BT, 8, 128), jnp.float32),
      scratch_types=[
          pltpu.VMEM((GPC, 128), jnp.int32),
          pltpu.VMEM((CHUNK, DIM), jnp.float32),
          pltpu.VMEM((2, GPC, 8, 128), jnp.float32),
          pltpu.SemaphoreType.DMA,
      ],
      compiler_params=pltpu.CompilerParams(
          use_tc_tiling_on_sc=False, needs_layout_passes=False),
  )
  return f(idx3d, flat_tab)


def kernel(value, tables):
  flat_tab = tables.reshape(N_COLS * VOCAB, DIM)
  offs = jnp.arange(N_COLS, dtype=jnp.int32)[:, None] * VOCAB
  idx3d = (value.astype(jnp.int32).T + offs).reshape(N_COLS, BT, 128)
  out5d = _embed(idx3d, flat_tab)
  return out5d.transpose(2, 4, 0, 1, 3).reshape(B, N_COLS, DIM)
